# baseline (speedup 1.0000x reference)
import functools

import jax
import jax.numpy as jnp
from jax import lax
from jax.experimental import pallas as pl
from jax.experimental.pallas import tpu as pltpu
from jax.experimental.pallas import tpu_sc as plsc

NUM_EXPERTS = 8
TOP_K = 2
NEG = -1e30
BLK = 512
G_MAX = 16
P = G_MAX * BLK
NW = 32
BF = 512


def _router_body(x_ref, gw_ref, logits_ref, tmi_ref, w0r_ref, w1r_ref,
                 bmeta_ref):
    x = x_ref[...]
    gw = gw_ref[...]
    logits = lax.dot_general(x, gw, (((1,), (1,)), ((), ())),
                             preferred_element_type=jnp.float32)
    logits_ref[...] = logits
    T = logits.shape[0]
    lane = lax.broadcasted_iota(jnp.int32, (T, NUM_EXPERTS), 1)
    m0 = jnp.max(logits, axis=1, keepdims=True)
    e0 = jnp.min(jnp.where(logits == m0, lane, NUM_EXPERTS), axis=1,
                 keepdims=True)
    masked = jnp.where(lane == e0, NEG, logits)
    m1 = jnp.max(masked, axis=1, keepdims=True)
    e1 = jnp.min(jnp.where(masked == m1, lane, NUM_EXPERTS), axis=1,
                 keepdims=True)
    w0 = jax.nn.sigmoid(m0 - m1)
    w1 = 1.0 - w0

    ohA = jnp.where(lane == e0, 1.0, 0.0)
    ohB = jnp.where(lane == e1, 1.0, 0.0)
    oh = ohA + ohB
    r_io = lax.broadcasted_iota(jnp.int32, (T, T), 0)
    c_io = lax.broadcasted_iota(jnp.int32, (T, T), 1)
    tri = jnp.where(r_io >= c_io, 1.0, 0.0)
    cum_incl = lax.dot_general(tri, oh, (((1,), (0,)), ((), ())),
                               preferred_element_type=jnp.float32)
    cum_excl = cum_incl - oh
    counts = cum_incl[T - 1:T, :]
    nb = jnp.floor((counts + (BLK - 1.0)) * (1.0 / BLK))
    e_io8 = lax.broadcasted_iota(jnp.int32, (NUM_EXPERTS, NUM_EXPERTS), 0)
    f_io8 = lax.broadcasted_iota(jnp.int32, (NUM_EXPERTS, NUM_EXPERTS), 1)
    tri8 = jnp.where(e_io8 < f_io8, 1.0, 0.0)
    offb = lax.dot_general(nb, tri8, (((1,), (0,)), ((), ())),
                           preferred_element_type=jnp.float32)
    off = offb * float(BLK)

    off_sel0 = jnp.sum(ohA * off, axis=1, keepdims=True)
    off_sel1 = jnp.sum(ohB * off, axis=1, keepdims=True)
    rank0 = jnp.sum(ohA * cum_excl, axis=1, keepdims=True)
    rank1 = jnp.sum(ohB * cum_excl, axis=1, keepdims=True)
    slot0 = (off_sel0 + rank0).astype(jnp.int32)
    slot1 = (off_sel1 + rank1).astype(jnp.int32)

    tmi_ref[...] = jnp.where(lane == 0, slot0, 0) + jnp.where(lane == 1, slot1, 0)
    w0r_ref[...] = jnp.broadcast_to(w0, (T, 16))
    w1r_ref[...] = jnp.broadcast_to(w1, (T, 16))

    lane128 = lax.broadcasted_iota(jnp.int32, (1, 128), 1)
    bexp = jnp.zeros((1, 128), jnp.float32)
    for e in range(NUM_EXPERTS):
        lo = off[:, e:e + 1] * (1.0 / BLK)
        hi = lo + nb[:, e:e + 1]
        g = lane128.astype(jnp.float32)
        in_e = jnp.where((g >= lo) & (g < hi), 1.0, 0.0)
        bexp = bexp + in_e * float(e)
    n_used = jnp.sum(nb, axis=1, keepdims=True)
    g_f = lane128.astype(jnp.float32)
    bexp_last = jnp.sum(jnp.where(g_f == n_used - 1.0, bexp, 0.0), axis=1,
                        keepdims=True)
    bexp = jnp.where(g_f < n_used, bexp, bexp_last)
    bmeta = (jnp.where(lane128 < G_MAX, bexp, 0.0)
             + jnp.where(lane128 == G_MAX, n_used, 0.0))
    bmeta_ref[...] = bmeta.astype(jnp.int32)


def _router_call(x, gate_w):
    T = x.shape[0]
    return pl.pallas_call(
        _router_body,
        out_shape=(
            jax.ShapeDtypeStruct((T, NUM_EXPERTS), jnp.float32),
            jax.ShapeDtypeStruct((T, NUM_EXPERTS), jnp.int32),
            jax.ShapeDtypeStruct((T, 16), jnp.float32),
            jax.ShapeDtypeStruct((T, 16), jnp.float32),
            jax.ShapeDtypeStruct((1, 128), jnp.int32),
        ),
    )(x, gate_w)


def _dispatch_call(xi, idx3):
    T, H2 = xi.shape
    tpw = T // NW
    mesh = plsc.VectorSubcoreMesh(core_axis_name="c", subcore_axis_name="s")

    @functools.partial(
        pl.kernel,
        mesh=mesh,
        out_type=jax.ShapeDtypeStruct((P, H2), jnp.int32),
        scratch_types=[
            pltpu.VMEM((tpw, H2), jnp.int32),
            pltpu.VMEM((tpw,), jnp.int32),
            pltpu.VMEM((tpw,), jnp.int32),
            pltpu.SemaphoreType.DMA,
            pltpu.SemaphoreType.DMA,
        ],
    )
    def k(x_hbm, idx_hbm, out_hbm, rows_v, i0_v, i1_v, sem0, sem1):
        wid = lax.axis_index("s") * 2 + lax.axis_index("c")
        base = wid * tpw
        pltpu.sync_copy(x_hbm.at[pl.ds(base, tpw)], rows_v)
        pltpu.sync_copy(idx_hbm.at[wid, 0], i0_v)
        pltpu.sync_copy(idx_hbm.at[wid, 1], i1_v)
        c0 = pltpu.async_copy(rows_v, out_hbm.at[i0_v], sem0)
        c1 = pltpu.async_copy(rows_v, out_hbm.at[i1_v], sem1)
        c0.wait()
        c1.wait()

    return k(xi, idx3)


def _ffn_body(bexp_ref, nblk_ref, x_ref, w1_ref, w3_ref, w2_ref, out_ref):
    g = pl.program_id(0)
    f = pl.program_id(1)

    @pl.when(jnp.logical_and(f == 0, g < nblk_ref[0]))
    def _():
        out_ref[...] = jnp.zeros_like(out_ref)

    @pl.when(g < nblk_ref[0])
    def _():
        xb = x_ref[...]
        a = lax.dot_general(xb, w1_ref[0], (((1,), (1,)), ((), ())),
                            preferred_element_type=jnp.float32)
        b = lax.dot_general(xb, w3_ref[0], (((1,), (1,)), ((), ())),
                            preferred_element_type=jnp.float32)
        h = (a * jax.nn.sigmoid(a)) * b
        o = lax.dot_general(h.astype(jnp.bfloat16), w2_ref[0],
                            (((1,), (1,)), ((), ())),
                            preferred_element_type=jnp.float32)
        out_ref[...] += o


def _ffn_call(xsb, w1b, w3b, w2b, bexp, nblk):
    H = w1b.shape[2]
    F = w1b.shape[1]
    grid_spec = pltpu.PrefetchScalarGridSpec(
        num_scalar_prefetch=2,
        grid=(G_MAX, F // BF),
        in_specs=[
            pl.BlockSpec((BLK, H),
                         lambda g, f, be, nb: (jnp.minimum(g, nb[0] - 1), 0)),
            pl.BlockSpec((1, BF, H),
                         lambda g, f, be, nb: (be[g],
                                               jnp.where(g < nb[0], f, F // BF - 1),
                                               0)),
            pl.BlockSpec((1, BF, H),
                         lambda g, f, be, nb: (be[g],
                                               jnp.where(g < nb[0], f, F // BF - 1),
                                               0)),
            pl.BlockSpec((1, H, BF),
                         lambda g, f, be, nb: (be[g], 0,
                                               jnp.where(g < nb[0], f, F // BF - 1))),
        ],
        out_specs=pl.BlockSpec((BLK, H),
                               lambda g, f, be, nb: (jnp.minimum(g, nb[0] - 1), 0)),
    )
    return pl.pallas_call(
        _ffn_body,
        grid_spec=grid_spec,
        out_shape=jax.ShapeDtypeStruct((P, H), jnp.float32),
    )(bexp, nblk, xsb, w1b, w3b, w2b)


def _combine_call(y, idx3, w0r, w1r, T):
    H = y.shape[1]
    tpw = T // NW
    CH = 32
    NCH = tpw // CH
    mesh = plsc.VectorSubcoreMesh(core_axis_name="c", subcore_axis_name="s")

    @functools.partial(
        pl.kernel,
        mesh=mesh,
        out_type=jax.ShapeDtypeStruct((T, H), jnp.float32),
        scratch_types=[
            pltpu.VMEM((tpw,), jnp.int32),
            pltpu.VMEM((tpw,), jnp.int32),
            pltpu.VMEM((tpw, 16), jnp.float32),
            pltpu.VMEM((tpw, 16), jnp.float32),
            pltpu.VMEM((CH, H), jnp.float32),
            pltpu.VMEM((CH, H), jnp.float32),
            pltpu.VMEM((CH, H), jnp.float32),
            pltpu.SemaphoreType.DMA,
            pltpu.SemaphoreType.DMA,
        ],
    )
    def k(y_hbm, idx_hbm, w0r_hbm, w1r_hbm, out_hbm,
          i0_v, i1_v, w0_v, w1_v, y0_v, y1_v, ob_v, sem0, sem1):
        wid = lax.axis_index("s") * 2 + lax.axis_index("c")
        base = wid * tpw
        pltpu.sync_copy(idx_hbm.at[wid, 0], i0_v)
        pltpu.sync_copy(idx_hbm.at[wid, 1], i1_v)
        pltpu.sync_copy(w0r_hbm.at[pl.ds(base, tpw)], w0_v)
        pltpu.sync_copy(w1r_hbm.at[pl.ds(base, tpw)], w1_v)
        for c in range(NCH):
            g0 = pltpu.async_copy(y_hbm.at[i0_v.at[pl.ds(c * CH, CH)]], y0_v,
                                  sem0)
            g1 = pltpu.async_copy(y_hbm.at[i1_v.at[pl.ds(c * CH, CH)]], y1_v,
                                  sem1)
            g0.wait()
            g1.wait()

            def body_j(j, carry):
                w0s = w0_v[c * CH + j, :]
                w1s = w1_v[c * CH + j, :]

                def body_v(v, __):
                    a = (y0_v[j, pl.ds(v * 16, 16)] * w0s
                         + y1_v[j, pl.ds(v * 16, 16)] * w1s)
                    ob_v[j, pl.ds(v * 16, 16)] = a
                    return __

                return lax.fori_loop(0, H // 16, body_v, carry)

            lax.fori_loop(0, CH, body_j, 0)
            pltpu.sync_copy(ob_v, out_hbm.at[pl.ds(base + c * CH, CH)])

    return k(y, idx3, w0r, w1r)


def kernel(hidden_states, gate_w, w1, w2, w3):
    B, S, H = hidden_states.shape
    T = B * S
    E, F, _ = w1.shape
    x = hidden_states.reshape(T, H)

    logits, tmi, w0r, w1r, bmeta = _router_call(x, gate_w)

    idx3 = tmi[:, :2].T.reshape(2, NW, T // NW).transpose(1, 0, 2)
    bexp = bmeta[0, :G_MAX]
    nblk = bmeta[0, G_MAX:G_MAX + 1]

    xi = lax.bitcast_convert_type(
        x.astype(jnp.bfloat16).reshape(T, H // 2, 2), jnp.int32)
    xsi = _dispatch_call(xi, idx3)
    xsb = lax.bitcast_convert_type(xsi, jnp.bfloat16).reshape(P, H)

    w1b = w1.astype(jnp.bfloat16)
    w3b = w3.astype(jnp.bfloat16)
    w2b = w2.astype(jnp.bfloat16)
    y = _ffn_call(xsb, w1b, w3b, w2b, bexp, nblk)

    final = _combine_call(y, idx3, w0r, w1r, T)
    return final.reshape(B, S, H), logits


# speedup vs baseline: 1.4082x; 1.4082x over previous
import functools

import jax
import jax.numpy as jnp
from jax import lax
from jax.experimental import pallas as pl
from jax.experimental.pallas import tpu as pltpu
from jax.experimental.pallas import tpu_sc as plsc

NUM_EXPERTS = 8
TOP_K = 2
NEG = -1e30
BLK = 512
G_MAX = 16
P = G_MAX * BLK
NW = 32
BF = 512


def _router_body(x_ref, gw_ref, logits_ref, tmi_ref, w0r_ref, w1r_ref,
                 bmeta_ref):
    x = x_ref[...]
    gw = gw_ref[...]
    logits = lax.dot_general(x, gw, (((1,), (1,)), ((), ())),
                             preferred_element_type=jnp.float32)
    logits_ref[...] = logits
    T = logits.shape[0]
    lane = lax.broadcasted_iota(jnp.int32, (T, NUM_EXPERTS), 1)
    m0 = jnp.max(logits, axis=1, keepdims=True)
    e0 = jnp.min(jnp.where(logits == m0, lane, NUM_EXPERTS), axis=1,
                 keepdims=True)
    masked = jnp.where(lane == e0, NEG, logits)
    m1 = jnp.max(masked, axis=1, keepdims=True)
    e1 = jnp.min(jnp.where(masked == m1, lane, NUM_EXPERTS), axis=1,
                 keepdims=True)
    w0 = jax.nn.sigmoid(m0 - m1)
    w1 = 1.0 - w0

    ohA = jnp.where(lane == e0, 1.0, 0.0)
    ohB = jnp.where(lane == e1, 1.0, 0.0)
    oh = ohA + ohB
    r_io = lax.broadcasted_iota(jnp.int32, (T, T), 0)
    c_io = lax.broadcasted_iota(jnp.int32, (T, T), 1)
    tri = jnp.where(r_io >= c_io, 1.0, 0.0)
    cum_incl = lax.dot_general(tri, oh, (((1,), (0,)), ((), ())),
                               preferred_element_type=jnp.float32)
    cum_excl = cum_incl - oh
    counts = cum_incl[T - 1:T, :]
    nb = jnp.floor((counts + (BLK - 1.0)) * (1.0 / BLK))
    e_io8 = lax.broadcasted_iota(jnp.int32, (NUM_EXPERTS, NUM_EXPERTS), 0)
    f_io8 = lax.broadcasted_iota(jnp.int32, (NUM_EXPERTS, NUM_EXPERTS), 1)
    tri8 = jnp.where(e_io8 < f_io8, 1.0, 0.0)
    offb = lax.dot_general(nb, tri8, (((1,), (0,)), ((), ())),
                           preferred_element_type=jnp.float32)
    off = offb * float(BLK)

    off_sel0 = jnp.sum(ohA * off, axis=1, keepdims=True)
    off_sel1 = jnp.sum(ohB * off, axis=1, keepdims=True)
    rank0 = jnp.sum(ohA * cum_excl, axis=1, keepdims=True)
    rank1 = jnp.sum(ohB * cum_excl, axis=1, keepdims=True)
    slot0 = (off_sel0 + rank0).astype(jnp.int32)
    slot1 = (off_sel1 + rank1).astype(jnp.int32)

    tmi_ref[...] = jnp.where(lane == 0, slot0, 0) + jnp.where(lane == 1, slot1, 0)
    w0r_ref[...] = jnp.broadcast_to(w0, (T, 16))
    w1r_ref[...] = jnp.broadcast_to(w1, (T, 16))

    lane128 = lax.broadcasted_iota(jnp.int32, (1, 128), 1)
    bexp = jnp.zeros((1, 128), jnp.float32)
    for e in range(NUM_EXPERTS):
        lo = off[:, e:e + 1] * (1.0 / BLK)
        hi = lo + nb[:, e:e + 1]
        g = lane128.astype(jnp.float32)
        in_e = jnp.where((g >= lo) & (g < hi), 1.0, 0.0)
        bexp = bexp + in_e * float(e)
    n_used = jnp.sum(nb, axis=1, keepdims=True)
    g_f = lane128.astype(jnp.float32)
    bexp_last = jnp.sum(jnp.where(g_f == n_used - 1.0, bexp, 0.0), axis=1,
                        keepdims=True)
    bexp = jnp.where(g_f < n_used, bexp, bexp_last)
    bmeta = (jnp.where(lane128 < G_MAX, bexp, 0.0)
             + jnp.where(lane128 == G_MAX, n_used, 0.0))
    bmeta_ref[...] = bmeta.astype(jnp.int32)


def _router_call(x, gate_w):
    T = x.shape[0]
    return pl.pallas_call(
        _router_body,
        out_shape=(
            jax.ShapeDtypeStruct((T, NUM_EXPERTS), jnp.float32),
            jax.ShapeDtypeStruct((T, NUM_EXPERTS), jnp.int32),
            jax.ShapeDtypeStruct((T, 16), jnp.float32),
            jax.ShapeDtypeStruct((T, 16), jnp.float32),
            jax.ShapeDtypeStruct((1, 128), jnp.int32),
        ),
    )(x, gate_w)


def _dispatch_call(xi, idx3):
    T, H2 = xi.shape
    tpw = T // NW
    mesh = plsc.VectorSubcoreMesh(core_axis_name="c", subcore_axis_name="s")

    @functools.partial(
        pl.kernel,
        mesh=mesh,
        out_type=jax.ShapeDtypeStruct((P, H2), jnp.float32),
        scratch_types=[
            pltpu.VMEM((tpw, H2), jnp.float32),
            pltpu.VMEM((tpw,), jnp.int32),
            pltpu.VMEM((tpw,), jnp.int32),
            pltpu.SemaphoreType.DMA,
            pltpu.SemaphoreType.DMA,
        ],
    )
    def k(x_hbm, idx_hbm, out_hbm, rows_v, i0_v, i1_v, sem0, sem1):
        wid = lax.axis_index("s") * 2 + lax.axis_index("c")
        base = wid * tpw
        pltpu.sync_copy(x_hbm.at[pl.ds(base, tpw)], rows_v)
        pltpu.sync_copy(idx_hbm.at[wid, 0], i0_v)
        pltpu.sync_copy(idx_hbm.at[wid, 1], i1_v)
        c0 = pltpu.async_copy(rows_v, out_hbm.at[i0_v], sem0)
        c1 = pltpu.async_copy(rows_v, out_hbm.at[i1_v], sem1)
        c0.wait()
        c1.wait()

    return k(xi, idx3)


def _ffn_body(bexp_ref, nblk_ref, x_ref, w1_ref, w3_ref, w2_ref, out_ref):
    g = pl.program_id(0)
    f = pl.program_id(1)

    @pl.when(jnp.logical_and(f == 0, g < nblk_ref[0]))
    def _():
        out_ref[...] = jnp.zeros_like(out_ref)

    @pl.when(g < nblk_ref[0])
    def _():
        xb = x_ref[...].astype(jnp.bfloat16)
        a = lax.dot_general(xb, w1_ref[0], (((1,), (1,)), ((), ())),
                            preferred_element_type=jnp.float32)
        b = lax.dot_general(xb, w3_ref[0], (((1,), (1,)), ((), ())),
                            preferred_element_type=jnp.float32)
        h = (a * jax.nn.sigmoid(a)) * b
        o = lax.dot_general(h.astype(jnp.bfloat16), w2_ref[0],
                            (((1,), (1,)), ((), ())),
                            preferred_element_type=jnp.float32)
        out_ref[...] += o


def _ffn_call(xsb, w1b, w3b, w2b, bexp, nblk):
    H = w1b.shape[2]
    F = w1b.shape[1]
    grid_spec = pltpu.PrefetchScalarGridSpec(
        num_scalar_prefetch=2,
        grid=(G_MAX, F // BF),
        in_specs=[
            pl.BlockSpec((BLK, H),
                         lambda g, f, be, nb: (jnp.minimum(g, nb[0] - 1), 0)),
            pl.BlockSpec((1, BF, H),
                         lambda g, f, be, nb: (be[g],
                                               jnp.where(g < nb[0], f, F // BF - 1),
                                               0)),
            pl.BlockSpec((1, BF, H),
                         lambda g, f, be, nb: (be[g],
                                               jnp.where(g < nb[0], f, F // BF - 1),
                                               0)),
            pl.BlockSpec((1, H, BF),
                         lambda g, f, be, nb: (be[g], 0,
                                               jnp.where(g < nb[0], f, F // BF - 1))),
        ],
        out_specs=pl.BlockSpec((BLK, H),
                               lambda g, f, be, nb: (jnp.minimum(g, nb[0] - 1), 0)),
    )
    return pl.pallas_call(
        _ffn_body,
        grid_spec=grid_spec,
        out_shape=jax.ShapeDtypeStruct((P, H), jnp.float32),
    )(bexp, nblk, xsb, w1b, w3b, w2b)


def _combine_call(y, idx3, w0r, w1r, T):
    H = y.shape[1]
    tpw = T // NW
    CH = 32
    NCH = tpw // CH
    mesh = plsc.VectorSubcoreMesh(core_axis_name="c", subcore_axis_name="s")

    @functools.partial(
        pl.kernel,
        mesh=mesh,
        out_type=jax.ShapeDtypeStruct((T, H), jnp.float32),
        scratch_types=[
            pltpu.VMEM((tpw,), jnp.int32),
            pltpu.VMEM((tpw,), jnp.int32),
            pltpu.VMEM((tpw, 16), jnp.float32),
            pltpu.VMEM((tpw, 16), jnp.float32),
            pltpu.VMEM((CH, H), jnp.float32),
            pltpu.VMEM((CH, H), jnp.float32),
            pltpu.VMEM((CH, H), jnp.float32),
            pltpu.SemaphoreType.DMA,
            pltpu.SemaphoreType.DMA,
        ],
    )
    def k(y_hbm, idx_hbm, w0r_hbm, w1r_hbm, out_hbm,
          i0_v, i1_v, w0_v, w1_v, y0_v, y1_v, ob_v, sem0, sem1):
        wid = lax.axis_index("s") * 2 + lax.axis_index("c")
        base = wid * tpw
        pltpu.sync_copy(idx_hbm.at[wid, 0], i0_v)
        pltpu.sync_copy(idx_hbm.at[wid, 1], i1_v)
        pltpu.sync_copy(w0r_hbm.at[pl.ds(base, tpw)], w0_v)
        pltpu.sync_copy(w1r_hbm.at[pl.ds(base, tpw)], w1_v)
        for c in range(NCH):
            g0 = pltpu.async_copy(y_hbm.at[i0_v.at[pl.ds(c * CH, CH)]], y0_v,
                                  sem0)
            g1 = pltpu.async_copy(y_hbm.at[i1_v.at[pl.ds(c * CH, CH)]], y1_v,
                                  sem1)
            g0.wait()
            g1.wait()

            def body_j(j, carry):
                w0s = w0_v[c * CH + j, :]
                w1s = w1_v[c * CH + j, :]

                def body_v(v, __):
                    a = (y0_v[j, pl.ds(v * 16, 16)] * w0s
                         + y1_v[j, pl.ds(v * 16, 16)] * w1s)
                    ob_v[j, pl.ds(v * 16, 16)] = a
                    return __

                return lax.fori_loop(0, H // 16, body_v, carry)

            lax.fori_loop(0, CH, body_j, 0)
            pltpu.sync_copy(ob_v, out_hbm.at[pl.ds(base + c * CH, CH)])

    return k(y, idx3, w0r, w1r)


def kernel(hidden_states, gate_w, w1, w2, w3):
    B, S, H = hidden_states.shape
    T = B * S
    E, F, _ = w1.shape
    x = hidden_states.reshape(T, H)

    logits, tmi, w0r, w1r, bmeta = _router_call(x, gate_w)

    idx3 = tmi[:, :2].T.reshape(2, NW, T // NW).transpose(1, 0, 2)
    bexp = bmeta[0, :G_MAX]
    nblk = bmeta[0, G_MAX:G_MAX + 1]

    xsb = _dispatch_call(x, idx3)

    w1b = w1.astype(jnp.bfloat16)
    w3b = w3.astype(jnp.bfloat16)
    w2b = w2.astype(jnp.bfloat16)
    y = _ffn_call(xsb, w1b, w3b, w2b, bexp, nblk)

    final = _combine_call(y, idx3, w0r, w1r, T)
    return final.reshape(B, S, H), logits


# speedup vs baseline: 2.0663x; 1.4673x over previous
import functools

import jax
import jax.numpy as jnp
from jax import lax
from jax.experimental import pallas as pl
from jax.experimental.pallas import tpu as pltpu
from jax.experimental.pallas import tpu_sc as plsc

NUM_EXPERTS = 8
TOP_K = 2
NEG = -1e30
BLK = 512
G_MAX = 16
P = G_MAX * BLK
NW = 32
BF = 512


def _router_body(x_ref, gw_ref, logits_ref, tmi_ref, w0r_ref, w1r_ref,
                 bmeta_ref):
    x = x_ref[...]
    gw = gw_ref[...]
    logits = lax.dot_general(x, gw, (((1,), (1,)), ((), ())),
                             preferred_element_type=jnp.float32)
    logits_ref[...] = logits
    T = logits.shape[0]
    lane = lax.broadcasted_iota(jnp.int32, (T, NUM_EXPERTS), 1)
    m0 = jnp.max(logits, axis=1, keepdims=True)
    e0 = jnp.min(jnp.where(logits == m0, lane, NUM_EXPERTS), axis=1,
                 keepdims=True)
    masked = jnp.where(lane == e0, NEG, logits)
    m1 = jnp.max(masked, axis=1, keepdims=True)
    e1 = jnp.min(jnp.where(masked == m1, lane, NUM_EXPERTS), axis=1,
                 keepdims=True)
    w0 = jax.nn.sigmoid(m0 - m1)
    w1 = 1.0 - w0

    ohA = jnp.where(lane == e0, 1.0, 0.0)
    ohB = jnp.where(lane == e1, 1.0, 0.0)
    oh = ohA + ohB
    r_io = lax.broadcasted_iota(jnp.int32, (T, T), 0)
    c_io = lax.broadcasted_iota(jnp.int32, (T, T), 1)
    tri = jnp.where(r_io >= c_io, 1.0, 0.0)
    cum_incl = lax.dot_general(tri, oh, (((1,), (0,)), ((), ())),
                               preferred_element_type=jnp.float32)
    cum_excl = cum_incl - oh
    counts = cum_incl[T - 1:T, :]
    nb = jnp.floor((counts + (BLK - 1.0)) * (1.0 / BLK))
    e_io8 = lax.broadcasted_iota(jnp.int32, (NUM_EXPERTS, NUM_EXPERTS), 0)
    f_io8 = lax.broadcasted_iota(jnp.int32, (NUM_EXPERTS, NUM_EXPERTS), 1)
    tri8 = jnp.where(e_io8 < f_io8, 1.0, 0.0)
    offb = lax.dot_general(nb, tri8, (((1,), (0,)), ((), ())),
                           preferred_element_type=jnp.float32)
    off = offb * float(BLK)

    off_sel0 = jnp.sum(ohA * off, axis=1, keepdims=True)
    off_sel1 = jnp.sum(ohB * off, axis=1, keepdims=True)
    rank0 = jnp.sum(ohA * cum_excl, axis=1, keepdims=True)
    rank1 = jnp.sum(ohB * cum_excl, axis=1, keepdims=True)
    slot0 = (off_sel0 + rank0).astype(jnp.int32)
    slot1 = (off_sel1 + rank1).astype(jnp.int32)

    tmi_ref[...] = jnp.where(lane == 0, slot0, 0) + jnp.where(lane == 1, slot1, 0)
    w0r_ref[...] = jnp.broadcast_to(w0, (T, 16))
    w1r_ref[...] = jnp.broadcast_to(w1, (T, 16))

    lane128 = lax.broadcasted_iota(jnp.int32, (1, 128), 1)
    bexp = jnp.zeros((1, 128), jnp.float32)
    for e in range(NUM_EXPERTS):
        lo = off[:, e:e + 1] * (1.0 / BLK)
        hi = lo + nb[:, e:e + 1]
        g = lane128.astype(jnp.float32)
        in_e = jnp.where((g >= lo) & (g < hi), 1.0, 0.0)
        bexp = bexp + in_e * float(e)
    n_used = jnp.sum(nb, axis=1, keepdims=True)
    g_f = lane128.astype(jnp.float32)
    bexp_last = jnp.sum(jnp.where(g_f == n_used - 1.0, bexp, 0.0), axis=1,
                        keepdims=True)
    bexp = jnp.where(g_f < n_used, bexp, bexp_last)
    bmeta = (jnp.where(lane128 < G_MAX, bexp, 0.0)
             + jnp.where(lane128 == G_MAX, n_used, 0.0))
    bmeta_ref[...] = bmeta.astype(jnp.int32)


def _router_call(x, gate_w):
    T = x.shape[0]
    return pl.pallas_call(
        _router_body,
        out_shape=(
            jax.ShapeDtypeStruct((T, NUM_EXPERTS), jnp.float32),
            jax.ShapeDtypeStruct((T, NUM_EXPERTS), jnp.int32),
            jax.ShapeDtypeStruct((T, 16), jnp.float32),
            jax.ShapeDtypeStruct((T, 16), jnp.float32),
            jax.ShapeDtypeStruct((1, 128), jnp.int32),
        ),
    )(x, gate_w)


def _dispatch_call(xi, idx3):
    T, H2 = xi.shape
    tpw = T // NW
    mesh = plsc.VectorSubcoreMesh(core_axis_name="c", subcore_axis_name="s")

    @functools.partial(
        pl.kernel,
        mesh=mesh,
        out_type=jax.ShapeDtypeStruct((P, H2), jnp.float32),
        scratch_types=[
            pltpu.VMEM((tpw, H2), jnp.float32),
            pltpu.VMEM((tpw,), jnp.int32),
            pltpu.VMEM((tpw,), jnp.int32),
            pltpu.SemaphoreType.DMA,
            pltpu.SemaphoreType.DMA,
        ],
    )
    def k(x_hbm, idx_hbm, out_hbm, rows_v, i0_v, i1_v, sem0, sem1):
        wid = lax.axis_index("s") * 2 + lax.axis_index("c")
        base = wid * tpw
        pltpu.sync_copy(x_hbm.at[pl.ds(base, tpw)], rows_v)
        pltpu.sync_copy(idx_hbm.at[wid, 0], i0_v)
        pltpu.sync_copy(idx_hbm.at[wid, 1], i1_v)
        c0 = pltpu.async_copy(rows_v, out_hbm.at[i0_v], sem0)
        c1 = pltpu.async_copy(rows_v, out_hbm.at[i1_v], sem1)
        c0.wait()
        c1.wait()

    return k(xi, idx3)


def _ffn_body(bexp_ref, nblk_ref, x_ref, w1_ref, w3_ref, w2_ref, out_ref):
    g = pl.program_id(0)
    f = pl.program_id(1)

    @pl.when(jnp.logical_and(f == 0, g < nblk_ref[0]))
    def _():
        out_ref[...] = jnp.zeros_like(out_ref)

    @pl.when(g < nblk_ref[0])
    def _():
        xb = x_ref[...].astype(jnp.bfloat16)
        a = lax.dot_general(xb, w1_ref[0].astype(jnp.bfloat16),
                            (((1,), (1,)), ((), ())),
                            preferred_element_type=jnp.float32)
        b = lax.dot_general(xb, w3_ref[0].astype(jnp.bfloat16),
                            (((1,), (1,)), ((), ())),
                            preferred_element_type=jnp.float32)
        h = (a * jax.nn.sigmoid(a)) * b
        o = lax.dot_general(h.astype(jnp.bfloat16),
                            w2_ref[0].astype(jnp.bfloat16),
                            (((1,), (1,)), ((), ())),
                            preferred_element_type=jnp.float32)
        out_ref[...] += o


def _ffn_call(xsb, w1b, w3b, w2b, bexp, nblk):
    H = w1b.shape[2]
    F = w1b.shape[1]
    grid_spec = pltpu.PrefetchScalarGridSpec(
        num_scalar_prefetch=2,
        grid=(G_MAX, F // BF),
        in_specs=[
            pl.BlockSpec((BLK, H),
                         lambda g, f, be, nb: (jnp.minimum(g, nb[0] - 1), 0)),
            pl.BlockSpec((1, BF, H),
                         lambda g, f, be, nb: (be[g],
                                               jnp.where(g < nb[0], f, F // BF - 1),
                                               0)),
            pl.BlockSpec((1, BF, H),
                         lambda g, f, be, nb: (be[g],
                                               jnp.where(g < nb[0], f, F // BF - 1),
                                               0)),
            pl.BlockSpec((1, H, BF),
                         lambda g, f, be, nb: (be[g], 0,
                                               jnp.where(g < nb[0], f, F // BF - 1))),
        ],
        out_specs=pl.BlockSpec((BLK, H),
                               lambda g, f, be, nb: (jnp.minimum(g, nb[0] - 1), 0)),
    )
    return pl.pallas_call(
        _ffn_body,
        grid_spec=grid_spec,
        out_shape=jax.ShapeDtypeStruct((P, H), jnp.float32),
    )(bexp, nblk, xsb, w1b, w3b, w2b)


def _combine_call(y, idx3, w0r, w1r, T):
    H = y.shape[1]
    tpw = T // NW
    CH = 32
    NCH = tpw // CH
    mesh = plsc.VectorSubcoreMesh(core_axis_name="c", subcore_axis_name="s")

    @functools.partial(
        pl.kernel,
        mesh=mesh,
        out_type=jax.ShapeDtypeStruct((T, H), jnp.float32),
        scratch_types=[
            pltpu.VMEM((tpw,), jnp.int32),
            pltpu.VMEM((tpw,), jnp.int32),
            pltpu.VMEM((tpw, 16), jnp.float32),
            pltpu.VMEM((tpw, 16), jnp.float32),
            pltpu.VMEM((CH, H), jnp.float32),
            pltpu.VMEM((CH, H), jnp.float32),
            pltpu.VMEM((CH, H), jnp.float32),
            pltpu.SemaphoreType.DMA,
            pltpu.SemaphoreType.DMA,
        ],
    )
    def k(y_hbm, idx_hbm, w0r_hbm, w1r_hbm, out_hbm,
          i0_v, i1_v, w0_v, w1_v, y0_v, y1_v, ob_v, sem0, sem1):
        wid = lax.axis_index("s") * 2 + lax.axis_index("c")
        base = wid * tpw
        pltpu.sync_copy(idx_hbm.at[wid, 0], i0_v)
        pltpu.sync_copy(idx_hbm.at[wid, 1], i1_v)
        pltpu.sync_copy(w0r_hbm.at[pl.ds(base, tpw)], w0_v)
        pltpu.sync_copy(w1r_hbm.at[pl.ds(base, tpw)], w1_v)
        for c in range(NCH):
            g0 = pltpu.async_copy(y_hbm.at[i0_v.at[pl.ds(c * CH, CH)]], y0_v,
                                  sem0)
            g1 = pltpu.async_copy(y_hbm.at[i1_v.at[pl.ds(c * CH, CH)]], y1_v,
                                  sem1)
            g0.wait()
            g1.wait()

            def body_j(j, carry):
                w0s = w0_v[c * CH + j, :]
                w1s = w1_v[c * CH + j, :]

                def body_v(v, __):
                    a = (y0_v[j, pl.ds(v * 16, 16)] * w0s
                         + y1_v[j, pl.ds(v * 16, 16)] * w1s)
                    ob_v[j, pl.ds(v * 16, 16)] = a
                    return __

                return lax.fori_loop(0, H // 16, body_v, carry)

            lax.fori_loop(0, CH, body_j, 0)
            pltpu.sync_copy(ob_v, out_hbm.at[pl.ds(base + c * CH, CH)])

    return k(y, idx3, w0r, w1r)


def kernel(hidden_states, gate_w, w1, w2, w3):
    B, S, H = hidden_states.shape
    T = B * S
    E, F, _ = w1.shape
    x = hidden_states.reshape(T, H)

    logits, tmi, w0r, w1r, bmeta = _router_call(x, gate_w)

    idx3 = tmi[:, :2].T.reshape(2, NW, T // NW).transpose(1, 0, 2)
    bexp = bmeta[0, :G_MAX]
    nblk = bmeta[0, G_MAX:G_MAX + 1]

    xsb = _dispatch_call(x, idx3)
    y = _ffn_call(xsb, w1, w3, w2, bexp, nblk)

    final = _combine_call(y, idx3, w0r, w1r, T)
    return final.reshape(B, S, H), logits


# speedup vs baseline: 2.4552x; 1.1882x over previous
import functools

import jax
import jax.numpy as jnp
from jax import lax
from jax.experimental import pallas as pl
from jax.experimental.pallas import tpu as pltpu
from jax.experimental.pallas import tpu_sc as plsc

NUM_EXPERTS = 8
TOP_K = 2
NEG = -1e30
BLK = 640
G_MAX = 14
P = G_MAX * BLK
NW = 32
BF = 512


def _router_body(x_ref, gw_ref, logits_ref, tmi_ref, w0r_ref, w1r_ref,
                 bmeta_ref):
    x = x_ref[...]
    gw = gw_ref[...]
    logits = lax.dot_general(x, gw, (((1,), (1,)), ((), ())),
                             preferred_element_type=jnp.float32)
    logits_ref[...] = logits
    T = logits.shape[0]
    lane = lax.broadcasted_iota(jnp.int32, (T, NUM_EXPERTS), 1)
    m0 = jnp.max(logits, axis=1, keepdims=True)
    e0 = jnp.min(jnp.where(logits == m0, lane, NUM_EXPERTS), axis=1,
                 keepdims=True)
    masked = jnp.where(lane == e0, NEG, logits)
    m1 = jnp.max(masked, axis=1, keepdims=True)
    e1 = jnp.min(jnp.where(masked == m1, lane, NUM_EXPERTS), axis=1,
                 keepdims=True)
    w0 = jax.nn.sigmoid(m0 - m1)
    w1 = 1.0 - w0

    ohA = jnp.where(lane == e0, 1.0, 0.0)
    ohB = jnp.where(lane == e1, 1.0, 0.0)
    oh = ohA + ohB
    r_io = lax.broadcasted_iota(jnp.int32, (T, T), 0)
    c_io = lax.broadcasted_iota(jnp.int32, (T, T), 1)
    tri = jnp.where(r_io >= c_io, 1.0, 0.0)
    cum_incl = lax.dot_general(tri, oh, (((1,), (0,)), ((), ())),
                               preferred_element_type=jnp.float32)
    cum_excl = cum_incl - oh
    counts = cum_incl[T - 1:T, :]
    nb = jnp.floor((counts + (BLK - 1.0)) * (1.0 / BLK))
    e_io8 = lax.broadcasted_iota(jnp.int32, (NUM_EXPERTS, NUM_EXPERTS), 0)
    f_io8 = lax.broadcasted_iota(jnp.int32, (NUM_EXPERTS, NUM_EXPERTS), 1)
    tri8 = jnp.where(e_io8 < f_io8, 1.0, 0.0)
    offb = lax.dot_general(nb, tri8, (((1,), (0,)), ((), ())),
                           preferred_element_type=jnp.float32)
    off = offb * float(BLK)

    off_sel0 = jnp.sum(ohA * off, axis=1, keepdims=True)
    off_sel1 = jnp.sum(ohB * off, axis=1, keepdims=True)
    rank0 = jnp.sum(ohA * cum_excl, axis=1, keepdims=True)
    rank1 = jnp.sum(ohB * cum_excl, axis=1, keepdims=True)
    slot0 = (off_sel0 + rank0).astype(jnp.int32)
    slot1 = (off_sel1 + rank1).astype(jnp.int32)

    tmi_ref[...] = jnp.where(lane == 0, slot0, 0) + jnp.where(lane == 1, slot1, 0)
    w0r_ref[...] = jnp.broadcast_to(w0, (T, 16))
    w1r_ref[...] = jnp.broadcast_to(w1, (T, 16))

    lane128 = lax.broadcasted_iota(jnp.int32, (1, 128), 1)
    bexp = jnp.zeros((1, 128), jnp.float32)
    for e in range(NUM_EXPERTS):
        lo = off[:, e:e + 1] * (1.0 / BLK)
        hi = lo + nb[:, e:e + 1]
        g = lane128.astype(jnp.float32)
        in_e = jnp.where((g >= lo) & (g < hi), 1.0, 0.0)
        bexp = bexp + in_e * float(e)
    n_used = jnp.sum(nb, axis=1, keepdims=True)
    g_f = lane128.astype(jnp.float32)
    bexp_last = jnp.sum(jnp.where(g_f == n_used - 1.0, bexp, 0.0), axis=1,
                        keepdims=True)
    bexp = jnp.where(g_f < n_used, bexp, bexp_last)
    bmeta = (jnp.where(lane128 < G_MAX, bexp, 0.0)
             + jnp.where(lane128 == G_MAX, n_used, 0.0))
    bmeta_ref[...] = bmeta.astype(jnp.int32)


def _router_call(x, gate_w):
    T = x.shape[0]
    return pl.pallas_call(
        _router_body,
        out_shape=(
            jax.ShapeDtypeStruct((T, NUM_EXPERTS), jnp.float32),
            jax.ShapeDtypeStruct((T, NUM_EXPERTS), jnp.int32),
            jax.ShapeDtypeStruct((T, 16), jnp.float32),
            jax.ShapeDtypeStruct((T, 16), jnp.float32),
            jax.ShapeDtypeStruct((1, 128), jnp.int32),
        ),
    )(x, gate_w)


def _dispatch_call(xi, idx3):
    T, H2 = xi.shape
    tpw = T // NW
    mesh = plsc.VectorSubcoreMesh(core_axis_name="c", subcore_axis_name="s")

    @functools.partial(
        pl.kernel,
        mesh=mesh,
        out_type=jax.ShapeDtypeStruct((P, H2), jnp.float32),
        scratch_types=[
            pltpu.VMEM((tpw, H2), jnp.float32),
            pltpu.VMEM((tpw,), jnp.int32),
            pltpu.VMEM((tpw,), jnp.int32),
            pltpu.SemaphoreType.DMA,
            pltpu.SemaphoreType.DMA,
        ],
    )
    def k(x_hbm, idx_hbm, out_hbm, rows_v, i0_v, i1_v, sem0, sem1):
        wid = lax.axis_index("s") * 2 + lax.axis_index("c")
        base = wid * tpw
        pltpu.sync_copy(x_hbm.at[pl.ds(base, tpw)], rows_v)
        pltpu.sync_copy(idx_hbm.at[wid, 0], i0_v)
        pltpu.sync_copy(idx_hbm.at[wid, 1], i1_v)
        c0 = pltpu.async_copy(rows_v, out_hbm.at[i0_v], sem0)
        c1 = pltpu.async_copy(rows_v, out_hbm.at[i1_v], sem1)
        c0.wait()
        c1.wait()

    return k(xi, idx3)


def _ffn_body(bexp_ref, nblk_ref, x_ref, w1_ref, w3_ref, w2_ref, out_ref):
    g = pl.program_id(0)
    f = pl.program_id(1)

    @pl.when(jnp.logical_and(f == 0, g < nblk_ref[0]))
    def _():
        out_ref[...] = jnp.zeros_like(out_ref)

    @pl.when(g < nblk_ref[0])
    def _():
        xb = x_ref[...].astype(jnp.bfloat16)
        a = lax.dot_general(xb, w1_ref[0].astype(jnp.bfloat16),
                            (((1,), (1,)), ((), ())),
                            preferred_element_type=jnp.float32)
        b = lax.dot_general(xb, w3_ref[0].astype(jnp.bfloat16),
                            (((1,), (1,)), ((), ())),
                            preferred_element_type=jnp.float32)
        h = (a * jax.nn.sigmoid(a)) * b
        o = lax.dot_general(h.astype(jnp.bfloat16),
                            w2_ref[0].astype(jnp.bfloat16),
                            (((1,), (1,)), ((), ())),
                            preferred_element_type=jnp.float32)
        out_ref[...] += o


def _ffn_call(xsb, w1b, w3b, w2b, bexp, nblk):
    H = w1b.shape[2]
    F = w1b.shape[1]
    grid_spec = pltpu.PrefetchScalarGridSpec(
        num_scalar_prefetch=2,
        grid=(G_MAX, F // BF),
        in_specs=[
            pl.BlockSpec((BLK, H),
                         lambda g, f, be, nb: (jnp.minimum(g, nb[0] - 1), 0)),
            pl.BlockSpec((1, BF, H),
                         lambda g, f, be, nb: (be[g],
                                               jnp.where(g < nb[0], f, F // BF - 1),
                                               0)),
            pl.BlockSpec((1, BF, H),
                         lambda g, f, be, nb: (be[g],
                                               jnp.where(g < nb[0], f, F // BF - 1),
                                               0)),
            pl.BlockSpec((1, H, BF),
                         lambda g, f, be, nb: (be[g], 0,
                                               jnp.where(g < nb[0], f, F // BF - 1))),
        ],
        out_specs=pl.BlockSpec((BLK, H),
                               lambda g, f, be, nb: (jnp.minimum(g, nb[0] - 1), 0)),
    )
    return pl.pallas_call(
        _ffn_body,
        grid_spec=grid_spec,
        out_shape=jax.ShapeDtypeStruct((P, H), jnp.float32),
    )(bexp, nblk, xsb, w1b, w3b, w2b)


def _combine_call(y, idx3, w0r, w1r, T):
    H = y.shape[1]
    tpw = T // NW
    CH = 32
    NCH = tpw // CH
    mesh = plsc.VectorSubcoreMesh(core_axis_name="c", subcore_axis_name="s")

    @functools.partial(
        pl.kernel,
        mesh=mesh,
        out_type=jax.ShapeDtypeStruct((T, H), jnp.float32),
        scratch_types=[
            pltpu.VMEM((tpw,), jnp.int32),
            pltpu.VMEM((tpw,), jnp.int32),
            pltpu.VMEM((tpw, 16), jnp.float32),
            pltpu.VMEM((tpw, 16), jnp.float32),
            pltpu.VMEM((CH, H), jnp.float32),
            pltpu.VMEM((CH, H), jnp.float32),
            pltpu.VMEM((CH, H), jnp.float32),
            pltpu.SemaphoreType.DMA,
            pltpu.SemaphoreType.DMA,
        ],
    )
    def k(y_hbm, idx_hbm, w0r_hbm, w1r_hbm, out_hbm,
          i0_v, i1_v, w0_v, w1_v, y0_v, y1_v, ob_v, sem0, sem1):
        wid = lax.axis_index("s") * 2 + lax.axis_index("c")
        base = wid * tpw
        pltpu.sync_copy(idx_hbm.at[wid, 0], i0_v)
        pltpu.sync_copy(idx_hbm.at[wid, 1], i1_v)
        pltpu.sync_copy(w0r_hbm.at[pl.ds(base, tpw)], w0_v)
        pltpu.sync_copy(w1r_hbm.at[pl.ds(base, tpw)], w1_v)
        for c in range(NCH):
            g0 = pltpu.async_copy(y_hbm.at[i0_v.at[pl.ds(c * CH, CH)]], y0_v,
                                  sem0)
            g1 = pltpu.async_copy(y_hbm.at[i1_v.at[pl.ds(c * CH, CH)]], y1_v,
                                  sem1)
            g0.wait()
            g1.wait()

            def body_j(j, carry):
                w0s = w0_v[c * CH + j, :]
                w1s = w1_v[c * CH + j, :]

                def body_v(v, __):
                    a = (y0_v[j, pl.ds(v * 16, 16)] * w0s
                         + y1_v[j, pl.ds(v * 16, 16)] * w1s)
                    ob_v[j, pl.ds(v * 16, 16)] = a
                    return __

                return lax.fori_loop(0, H // 16, body_v, carry)

            lax.fori_loop(0, CH, body_j, 0)
            pltpu.sync_copy(ob_v, out_hbm.at[pl.ds(base + c * CH, CH)])

    return k(y, idx3, w0r, w1r)


def kernel(hidden_states, gate_w, w1, w2, w3):
    B, S, H = hidden_states.shape
    T = B * S
    E, F, _ = w1.shape
    x = hidden_states.reshape(T, H)

    logits, tmi, w0r, w1r, bmeta = _router_call(x, gate_w)

    idx3 = tmi[:, :2].T.reshape(2, NW, T // NW).transpose(1, 0, 2)
    bexp = bmeta[0, :G_MAX]
    nblk = bmeta[0, G_MAX:G_MAX + 1]

    xsb = _dispatch_call(x, idx3)
    y = _ffn_call(xsb, w1, w3, w2, bexp, nblk)

    final = _combine_call(y, idx3, w0r, w1r, T)
    return final.reshape(B, S, H), logits


# speedup vs baseline: 2.5566x; 1.0413x over previous
import functools

import jax
import jax.numpy as jnp
from jax import lax
from jax.experimental import pallas as pl
from jax.experimental.pallas import tpu as pltpu
from jax.experimental.pallas import tpu_sc as plsc

NUM_EXPERTS = 8
TOP_K = 2
NEG = -1e30
BLK = 640
G_MAX = 14
P = G_MAX * BLK
NW = 32
BF = 512


def _router_body(x_ref, gw_ref, logits_ref, tmi_ref, w0r_ref, w1r_ref,
                 bmeta_ref):
    x = x_ref[...]
    gw = gw_ref[...]
    logits = lax.dot_general(x, gw, (((1,), (1,)), ((), ())),
                             preferred_element_type=jnp.float32)
    logits_ref[...] = logits
    T = logits.shape[0]
    lane = lax.broadcasted_iota(jnp.int32, (T, NUM_EXPERTS), 1)
    m0 = jnp.max(logits, axis=1, keepdims=True)
    e0 = jnp.min(jnp.where(logits == m0, lane, NUM_EXPERTS), axis=1,
                 keepdims=True)
    masked = jnp.where(lane == e0, NEG, logits)
    m1 = jnp.max(masked, axis=1, keepdims=True)
    e1 = jnp.min(jnp.where(masked == m1, lane, NUM_EXPERTS), axis=1,
                 keepdims=True)
    w0 = jax.nn.sigmoid(m0 - m1)
    w1 = 1.0 - w0

    ohA = jnp.where(lane == e0, 1.0, 0.0)
    ohB = jnp.where(lane == e1, 1.0, 0.0)
    oh = ohA + ohB
    r_io = lax.broadcasted_iota(jnp.int32, (T, T), 0)
    c_io = lax.broadcasted_iota(jnp.int32, (T, T), 1)
    tri = jnp.where(r_io >= c_io, 1.0, 0.0)
    cum_incl = lax.dot_general(tri, oh, (((1,), (0,)), ((), ())),
                               preferred_element_type=jnp.float32)
    cum_excl = cum_incl - oh
    counts = cum_incl[T - 1:T, :]
    nb = jnp.floor((counts + (BLK - 1.0)) * (1.0 / BLK))
    e_io8 = lax.broadcasted_iota(jnp.int32, (NUM_EXPERTS, NUM_EXPERTS), 0)
    f_io8 = lax.broadcasted_iota(jnp.int32, (NUM_EXPERTS, NUM_EXPERTS), 1)
    tri8 = jnp.where(e_io8 < f_io8, 1.0, 0.0)
    offb = lax.dot_general(nb, tri8, (((1,), (0,)), ((), ())),
                           preferred_element_type=jnp.float32)
    off = offb * float(BLK)

    off_sel0 = jnp.sum(ohA * off, axis=1, keepdims=True)
    off_sel1 = jnp.sum(ohB * off, axis=1, keepdims=True)
    rank0 = jnp.sum(ohA * cum_excl, axis=1, keepdims=True)
    rank1 = jnp.sum(ohB * cum_excl, axis=1, keepdims=True)
    slot0 = (off_sel0 + rank0).astype(jnp.int32)
    slot1 = (off_sel1 + rank1).astype(jnp.int32)

    tmi_ref[...] = jnp.where(lane == 0, slot0, 0) + jnp.where(lane == 1, slot1, 0)
    w0r_ref[...] = jnp.broadcast_to(w0, (T, 16))
    w1r_ref[...] = jnp.broadcast_to(w1, (T, 16))

    lane128 = lax.broadcasted_iota(jnp.int32, (1, 128), 1)
    bexp = jnp.zeros((1, 128), jnp.float32)
    for e in range(NUM_EXPERTS):
        lo = off[:, e:e + 1] * (1.0 / BLK)
        hi = lo + nb[:, e:e + 1]
        g = lane128.astype(jnp.float32)
        in_e = jnp.where((g >= lo) & (g < hi), 1.0, 0.0)
        bexp = bexp + in_e * float(e)
    n_used = jnp.sum(nb, axis=1, keepdims=True)
    g_f = lane128.astype(jnp.float32)
    bexp_last = jnp.sum(jnp.where(g_f == n_used - 1.0, bexp, 0.0), axis=1,
                        keepdims=True)
    bexp = jnp.where(g_f < n_used, bexp, bexp_last)
    bmeta = (jnp.where(lane128 < G_MAX, bexp, 0.0)
             + jnp.where(lane128 == G_MAX, n_used, 0.0))
    bmeta_ref[...] = bmeta.astype(jnp.int32)


def _router_call(x, gate_w):
    T = x.shape[0]
    return pl.pallas_call(
        _router_body,
        out_shape=(
            jax.ShapeDtypeStruct((T, NUM_EXPERTS), jnp.float32),
            jax.ShapeDtypeStruct((T, NUM_EXPERTS), jnp.int32),
            jax.ShapeDtypeStruct((T, 16), jnp.float32),
            jax.ShapeDtypeStruct((T, 16), jnp.float32),
            jax.ShapeDtypeStruct((1, 128), jnp.int32),
        ),
    )(x, gate_w)


def _dispatch_call(xi, idx3):
    T, H2 = xi.shape
    tpw = T // NW
    mesh = plsc.VectorSubcoreMesh(core_axis_name="c", subcore_axis_name="s")

    @functools.partial(
        pl.kernel,
        mesh=mesh,
        out_type=jax.ShapeDtypeStruct((P, H2), jnp.float32),
        scratch_types=[
            pltpu.VMEM((tpw, H2), jnp.float32),
            pltpu.VMEM((tpw,), jnp.int32),
            pltpu.VMEM((tpw,), jnp.int32),
            pltpu.SemaphoreType.DMA,
            pltpu.SemaphoreType.DMA,
        ],
    )
    def k(x_hbm, idx_hbm, out_hbm, rows_v, i0_v, i1_v, sem0, sem1):
        wid = lax.axis_index("s") * 2 + lax.axis_index("c")
        base = wid * tpw
        pltpu.sync_copy(x_hbm.at[pl.ds(base, tpw)], rows_v)
        pltpu.sync_copy(idx_hbm.at[wid, 0], i0_v)
        pltpu.sync_copy(idx_hbm.at[wid, 1], i1_v)
        c0 = pltpu.async_copy(rows_v, out_hbm.at[i0_v], sem0)
        c1 = pltpu.async_copy(rows_v, out_hbm.at[i1_v], sem1)
        c0.wait()
        c1.wait()

    return k(xi, idx3)


def _ffn_body(bexp_ref, nblk_ref, x_ref, w1_ref, w3_ref, w2_ref, out_ref):
    g = pl.program_id(0)
    f = pl.program_id(1)

    @pl.when(jnp.logical_and(f == 0, g < nblk_ref[0]))
    def _():
        out_ref[...] = jnp.zeros_like(out_ref)

    @pl.when(g < nblk_ref[0])
    def _():
        xb = x_ref[...].astype(jnp.bfloat16)
        a = lax.dot_general(xb, w1_ref[0].astype(jnp.bfloat16),
                            (((1,), (1,)), ((), ())),
                            preferred_element_type=jnp.float32)
        b = lax.dot_general(xb, w3_ref[0].astype(jnp.bfloat16),
                            (((1,), (1,)), ((), ())),
                            preferred_element_type=jnp.float32)
        h = (a * jax.nn.sigmoid(a)) * b
        o = lax.dot_general(h.astype(jnp.bfloat16),
                            w2_ref[0].astype(jnp.bfloat16),
                            (((1,), (1,)), ((), ())),
                            preferred_element_type=jnp.float32)
        out_ref[...] += o


def _ffn_call(xsb, w1b, w3b, w2b, bexp, nblk):
    H = w1b.shape[2]
    F = w1b.shape[1]
    grid_spec = pltpu.PrefetchScalarGridSpec(
        num_scalar_prefetch=2,
        grid=(G_MAX, F // BF),
        in_specs=[
            pl.BlockSpec((BLK, H),
                         lambda g, f, be, nb: (jnp.minimum(g, nb[0] - 1), 0)),
            pl.BlockSpec((1, BF, H),
                         lambda g, f, be, nb: (be[g],
                                               jnp.where(g < nb[0], f, F // BF - 1),
                                               0)),
            pl.BlockSpec((1, BF, H),
                         lambda g, f, be, nb: (be[g],
                                               jnp.where(g < nb[0], f, F // BF - 1),
                                               0)),
            pl.BlockSpec((1, H, BF),
                         lambda g, f, be, nb: (be[g], 0,
                                               jnp.where(g < nb[0], f, F // BF - 1))),
        ],
        out_specs=pl.BlockSpec((BLK, H),
                               lambda g, f, be, nb: (jnp.minimum(g, nb[0] - 1), 0)),
    )
    return pl.pallas_call(
        _ffn_body,
        grid_spec=grid_spec,
        out_shape=jax.ShapeDtypeStruct((P, H), jnp.float32),
    )(bexp, nblk, xsb, w1b, w3b, w2b)


def _combine_call(y, idx3, w0r, w1r, T):
    H = y.shape[1]
    tpw = T // NW
    CH = 16
    NCH = tpw // CH
    mesh = plsc.VectorSubcoreMesh(core_axis_name="c", subcore_axis_name="s")

    @functools.partial(
        pl.kernel,
        mesh=mesh,
        out_type=jax.ShapeDtypeStruct((T, H), jnp.float32),
        scratch_types=[
            pltpu.VMEM((tpw,), jnp.int32),
            pltpu.VMEM((tpw,), jnp.int32),
            pltpu.VMEM((tpw, 16), jnp.float32),
            pltpu.VMEM((tpw, 16), jnp.float32),
            pltpu.VMEM((2, CH, H), jnp.float32),
            pltpu.VMEM((2, CH, H), jnp.float32),
            pltpu.VMEM((CH, H), jnp.float32),
            pltpu.SemaphoreType.DMA,
            pltpu.SemaphoreType.DMA,
            pltpu.SemaphoreType.DMA,
            pltpu.SemaphoreType.DMA,
        ],
    )
    def k(y_hbm, idx_hbm, w0r_hbm, w1r_hbm, out_hbm,
          i0_v, i1_v, w0_v, w1_v, y0_v, y1_v, ob_v, s00, s01, s10, s11):
        wid = lax.axis_index("s") * 2 + lax.axis_index("c")
        base = wid * tpw
        pltpu.sync_copy(idx_hbm.at[wid, 0], i0_v)
        pltpu.sync_copy(idx_hbm.at[wid, 1], i1_v)
        pltpu.sync_copy(w0r_hbm.at[pl.ds(base, tpw)], w0_v)
        pltpu.sync_copy(w1r_hbm.at[pl.ds(base, tpw)], w1_v)
        sems = ((s00, s10), (s01, s11))

        def issue(c, b):
            g0 = pltpu.async_copy(y_hbm.at[i0_v.at[pl.ds(c * CH, CH)]],
                                  y0_v.at[b], sems[b][0])
            g1 = pltpu.async_copy(y_hbm.at[i1_v.at[pl.ds(c * CH, CH)]],
                                  y1_v.at[b], sems[b][1])
            return g0, g1

        pending = {0: issue(0, 0)}
        for c in range(NCH):
            b = c % 2
            if c + 1 < NCH:
                pending[c + 1] = issue(c + 1, 1 - b)
            g0, g1 = pending.pop(c)
            g0.wait()
            g1.wait()
            yb0 = y0_v.at[b]
            yb1 = y1_v.at[b]

            def body_j(j, carry, c=c, yb0=yb0, yb1=yb1):
                w0s = w0_v[c * CH + j, :]
                w1s = w1_v[c * CH + j, :]
                for v in range(H // 16):
                    a = (yb0[j, pl.ds(v * 16, 16)] * w0s
                         + yb1[j, pl.ds(v * 16, 16)] * w1s)
                    ob_v[j, pl.ds(v * 16, 16)] = a
                return carry

            lax.fori_loop(0, CH, body_j, 0)
            pltpu.sync_copy(ob_v, out_hbm.at[pl.ds(base + c * CH, CH)])

    return k(y, idx3, w0r, w1r)


def kernel(hidden_states, gate_w, w1, w2, w3):
    B, S, H = hidden_states.shape
    T = B * S
    E, F, _ = w1.shape
    x = hidden_states.reshape(T, H)

    logits, tmi, w0r, w1r, bmeta = _router_call(x, gate_w)

    idx3 = tmi[:, :2].T.reshape(2, NW, T // NW).transpose(1, 0, 2)
    bexp = bmeta[0, :G_MAX]
    nblk = bmeta[0, G_MAX:G_MAX + 1]

    xsb = _dispatch_call(x, idx3)
    y = _ffn_call(xsb, w1, w3, w2, bexp, nblk)

    final = _combine_call(y, idx3, w0r, w1r, T)
    return final.reshape(B, S, H), logits
